# Initial kernel scaffold; baseline (speedup 1.0000x reference)
import jax
import jax.numpy as jnp
from jax.experimental import pallas as pl


def kernel(x, W_score, b_score):
    raise NotImplementedError("write your pallas kernel here")


# speedup vs baseline: 5.0638x; 5.0638x over previous
import numpy as np
import jax
import jax.numpy as jnp
from jax.experimental import pallas as pl
from jax.experimental.pallas import tpu as pltpu

_B, _N, _C = 8, 1568, 384
_CN = 196
_K = 9
_RC = 224
_SQRT_C = np.float32(np.sqrt(np.float32(_C)))


def _dist(dblock, sq_rows, sq_cols):
    d2 = (sq_rows + sq_cols) - 2.0 * dblock
    return jnp.sqrt(jnp.maximum(d2, 0.0)) / _SQRT_C


def _ctm_kernel(x_ref, w_ref, b_ref, noise_ref, out_ref,
                d_scr, sq_scr, dens_scr, score_scr, dg_scr, sqc_scr):
    xb = x_ref[0]

    sq_col = jnp.sum(xb * xb, axis=1, keepdims=True)
    sq_scr[...] = sq_col
    sq_row = sq_col.reshape(1, _N)

    d_scr[...] = jax.lax.dot_general(
        xb, xb, (((1,), (1,)), ((), ())),
        preferred_element_type=jnp.float32)

    n_chunks = _N // _RC
    col_iota = jax.lax.broadcasted_iota(jnp.int32, (_RC, _N), 1)

    def p1_body(c, dist_max):
        r0 = c * _RC
        sq_chunk = sq_scr[pl.ds(r0, _RC), :]
        dc = _dist(d_scr[pl.ds(r0, _RC), :], sq_chunk, sq_row)
        dist_max = jnp.maximum(dist_max, jnp.max(dc))

        def knn_body(_, carry):
            y, acc = carry
            m = jnp.min(y, axis=1, keepdims=True)
            am = jnp.min(jnp.where(y == m, col_iota, _N),
                         axis=1, keepdims=True)
            acc = acc + m * m
            y = jnp.where(col_iota == am, jnp.float32(jnp.inf), y)
            return y, acc

        _, acc = jax.lax.fori_loop(
            0, _K, knn_body, (dc, jnp.zeros((_RC, 1), jnp.float32)))
        dens_scr[pl.ds(r0, _RC), :] = jnp.exp(-(acc / _K))
        return dist_max

    dist_max = jax.lax.fori_loop(0, n_chunks, p1_body, jnp.float32(-jnp.inf))

    dens_scr[...] = dens_scr[...] + noise_ref[0].reshape(_N, 1)
    dens_row = jnp.transpose(dens_scr[...])

    def p2_body(c, _):
        r0 = c * _RC
        sq_chunk = sq_scr[pl.ds(r0, _RC), :]
        dc = _dist(d_scr[pl.ds(r0, _RC), :], sq_chunk, sq_row)
        dens_chunk = dens_scr[pl.ds(r0, _RC), :]
        cond = dens_row > dens_chunk
        delta = jnp.min(jnp.where(cond, dc, dist_max), axis=1, keepdims=True)
        score_scr[pl.ds(r0, _RC), :] = delta * dens_chunk
        return 0

    jax.lax.fori_loop(0, n_chunks, p2_body, 0)

    lane_iota = jax.lax.broadcasted_iota(jnp.int32, (1, _N), 1)
    score_row = jnp.transpose(score_scr[...])

    def topk_body(j, carry):
        sc, rank = carry
        m = jnp.max(sc)
        idx = jnp.min(jnp.where(sc == m, lane_iota, _N))
        dg_scr[pl.ds(j, 1), :] = d_scr[pl.ds(idx, 1), :]
        sqc_scr[pl.ds(j, 1), :] = sq_scr[pl.ds(idx, 1), :]
        rank = jnp.where(lane_iota == idx, j, rank)
        sc = jnp.where(lane_iota == idx, -jnp.float32(jnp.inf), sc)
        return sc, rank

    rank0 = jnp.full((1, _N), _CN, jnp.int32)
    _, rank = jax.lax.fori_loop(0, _CN, topk_body, (score_row, rank0))

    dist_g = _dist(dg_scr[...], sqc_scr[...], sq_row)
    dmin = jnp.min(dist_g, axis=0, keepdims=True)
    riota = jax.lax.broadcasted_iota(jnp.int32, (_CN, _N), 0)
    bj = jnp.min(jnp.where(dist_g == dmin, riota, _CN), axis=0, keepdims=True)
    clu = jnp.where(rank < _CN, rank, bj)

    wt_row = jnp.exp(jax.lax.dot_general(
        w_ref[...], xb, (((1,), (1,)), ((), ())),
        preferred_element_type=jnp.float32) + b_ref[0, 0])
    at = (riota == clu).astype(jnp.float32)
    aw = jax.lax.dot_general(
        wt_row, at, (((1,), (1,)), ((), ())),
        preferred_element_type=jnp.float32) + 1e-6
    aw_tok = jax.lax.dot_general(
        aw, at, (((1,), (0,)), ((), ())),
        preferred_element_type=jnp.float32)
    norm = wt_row / aw_tok
    out_ref[0] = jax.lax.dot_general(
        at * norm, xb, (((1,), (0,)), ((), ())),
        preferred_element_type=jnp.float32)


def kernel(x, W_score, b_score):
    noise = jax.random.uniform(jax.random.key(1), (_B, _N),
                               dtype=jnp.float32) * 1e-06
    return pl.pallas_call(
        _ctm_kernel,
        grid=(_B,),
        in_specs=[
            pl.BlockSpec((1, _N, _C), lambda i: (i, 0, 0)),
            pl.BlockSpec((1, _C), lambda i: (0, 0)),
            pl.BlockSpec((1, 1), lambda i: (0, 0)),
            pl.BlockSpec((1, 1, _N), lambda i: (i, 0, 0)),
        ],
        out_specs=pl.BlockSpec((1, _CN, _C), lambda i: (i, 0, 0)),
        out_shape=jax.ShapeDtypeStruct((_B, _CN, _C), jnp.float32),
        scratch_shapes=[
            pltpu.VMEM((_N, _N), jnp.float32),
            pltpu.VMEM((_N, 1), jnp.float32),
            pltpu.VMEM((_N, 1), jnp.float32),
            pltpu.VMEM((_N, 1), jnp.float32),
            pltpu.VMEM((_CN, _N), jnp.float32),
            pltpu.VMEM((_CN, 1), jnp.float32),
        ],
    )(x, W_score, b_score.reshape(1, 1), noise.reshape(_B, 1, _N))


# speedup vs baseline: 5.0646x; 1.0002x over previous
import numpy as np
import jax
import jax.numpy as jnp
from jax.experimental import pallas as pl
from jax.experimental.pallas import tpu as pltpu

_B, _N, _C = 8, 1568, 384
_CN = 196
_K = 9
_RC = 224
_SQRT_C = np.float32(np.sqrt(np.float32(_C)))


def _dist(dblock, sq_rows, sq_cols):
    d2 = (sq_rows + sq_cols) - 2.0 * dblock
    return jnp.sqrt(jnp.maximum(d2, 0.0)) / _SQRT_C


def _ctm_kernel(x_ref, w_ref, b_ref, noise_ref, out_ref,
                d_scr, sq_scr, dens_scr, score_scr, dg_scr, sqc_scr):
    xb = x_ref[0]

    sq_col = jnp.sum(xb * xb, axis=1, keepdims=True)
    sq_scr[...] = sq_col
    sq_row = sq_col.reshape(1, _N)

    d_scr[...] = jax.lax.dot_general(
        xb, xb, (((1,), (1,)), ((), ())),
        preferred_element_type=jnp.float32)

    n_chunks = _N // _RC
    col_iota = jax.lax.broadcasted_iota(jnp.int32, (_RC, _N), 1)

    def p1_body(c, dist_max):
        r0 = c * _RC
        sq_chunk = sq_scr[pl.ds(r0, _RC), :]
        dc = _dist(d_scr[pl.ds(r0, _RC), :], sq_chunk, sq_row)
        dist_max = jnp.maximum(dist_max, jnp.max(dc))

        def knn_body(_, carry):
            y, acc = carry
            m = jnp.min(y, axis=1, keepdims=True)
            am = jnp.min(jnp.where(y == m, col_iota, _N),
                         axis=1, keepdims=True)
            acc = acc + m * m
            y = jnp.where(col_iota == am, jnp.float32(jnp.inf), y)
            return y, acc

        _, acc = jax.lax.fori_loop(
            0, _K, knn_body, (dc, jnp.zeros((_RC, 1), jnp.float32)))
        dens_scr[pl.ds(r0, _RC), :] = jnp.exp(-(acc / _K))
        return dist_max

    dist_max = jax.lax.fori_loop(0, n_chunks, p1_body, jnp.float32(-jnp.inf))

    dens_scr[...] = dens_scr[...] + noise_ref[0].reshape(_N, 1)
    dens_row = jnp.transpose(dens_scr[...])

    def p2_body(c, _):
        r0 = c * _RC
        sq_chunk = sq_scr[pl.ds(r0, _RC), :]
        dc = _dist(d_scr[pl.ds(r0, _RC), :], sq_chunk, sq_row)
        dens_chunk = dens_scr[pl.ds(r0, _RC), :]
        cond = dens_row > dens_chunk
        delta = jnp.min(jnp.where(cond, dc, dist_max), axis=1, keepdims=True)
        score_scr[pl.ds(r0, _RC), :] = delta * dens_chunk
        return 0

    jax.lax.fori_loop(0, n_chunks, p2_body, 0)

    lane_iota = jax.lax.broadcasted_iota(jnp.int32, (1, _N), 1)
    score_row = jnp.transpose(score_scr[...])

    def topk_body(j, carry):
        sc, rank = carry
        m = jnp.max(sc)
        idx = jnp.min(jnp.where(sc == m, lane_iota, _N))
        dg_scr[pl.ds(j, 1), :] = d_scr[pl.ds(idx, 1), :]
        sqc_scr[pl.ds(j, 1), :] = sq_scr[pl.ds(idx, 1), :]
        rank = jnp.where(lane_iota == idx, j, rank)
        sc = jnp.where(lane_iota == idx, -jnp.float32(jnp.inf), sc)
        return sc, rank

    rank0 = jnp.full((1, _N), _CN, jnp.int32)
    _, rank = jax.lax.fori_loop(0, _CN, topk_body, (score_row, rank0))

    dist_g = _dist(dg_scr[...], sqc_scr[...], sq_row)
    dmin = jnp.min(dist_g, axis=0, keepdims=True)
    riota = jax.lax.broadcasted_iota(jnp.int32, (_CN, _N), 0)
    bj = jnp.min(jnp.where(dist_g == dmin, riota, _CN), axis=0, keepdims=True)
    clu = jnp.where(rank < _CN, rank, bj)

    wt_row = jnp.exp(jax.lax.dot_general(
        w_ref[...], xb, (((1,), (1,)), ((), ())),
        preferred_element_type=jnp.float32) + b_ref[0, 0])
    at = (riota == clu).astype(jnp.float32)
    aw = jax.lax.dot_general(
        wt_row, at, (((1,), (1,)), ((), ())),
        preferred_element_type=jnp.float32) + 1e-6
    aw_tok = jax.lax.dot_general(
        aw, at, (((1,), (0,)), ((), ())),
        preferred_element_type=jnp.float32)
    norm = wt_row / aw_tok
    out_ref[0] = jax.lax.dot_general(
        at * norm, xb, (((1,), (0,)), ((), ())),
        preferred_element_type=jnp.float32)


def kernel(x, W_score, b_score):
    noise = jax.random.uniform(jax.random.key(1), (_B, _N),
                               dtype=jnp.float32) * 1e-06
    return pl.pallas_call(
        _ctm_kernel,
        grid=(_B,),
        in_specs=[
            pl.BlockSpec((1, _N, _C), lambda i: (i, 0, 0)),
            pl.BlockSpec((1, _C), lambda i: (0, 0)),
            pl.BlockSpec((1, 1), lambda i: (0, 0)),
            pl.BlockSpec((1, 1, _N), lambda i: (i, 0, 0)),
        ],
        compiler_params=pltpu.CompilerParams(
            dimension_semantics=("parallel",)),
        out_specs=pl.BlockSpec((1, _CN, _C), lambda i: (i, 0, 0)),
        out_shape=jax.ShapeDtypeStruct((_B, _CN, _C), jnp.float32),
        scratch_shapes=[
            pltpu.VMEM((_N, _N), jnp.float32),
            pltpu.VMEM((_N, 1), jnp.float32),
            pltpu.VMEM((_N, 1), jnp.float32),
            pltpu.VMEM((_N, 1), jnp.float32),
            pltpu.VMEM((_CN, _N), jnp.float32),
            pltpu.VMEM((_CN, 1), jnp.float32),
        ],
    )(x, W_score, b_score.reshape(1, 1), noise.reshape(_B, 1, _N))


# speedup vs baseline: 5.1799x; 1.0228x over previous
import numpy as np
import jax
import jax.numpy as jnp
from jax.experimental import pallas as pl
from jax.experimental.pallas import tpu as pltpu

_B, _N, _C = 8, 1568, 384
_CN = 196
_K = 9
_RC = 224
_SQRT_C = np.float32(np.sqrt(np.float32(_C)))


def _d2(dblock, sq_rows, sq_cols):
    return jnp.maximum((sq_rows + sq_cols) - 2.0 * dblock, 0.0)


def _dist(dblock, sq_rows, sq_cols):
    return jnp.sqrt(_d2(dblock, sq_rows, sq_cols)) / _SQRT_C


def _ctm_kernel(x_ref, w_ref, b_ref, noise_ref, out_ref,
                d_scr, sq_scr, dens_scr, score_scr):
    xb = x_ref[0]

    sq_col = jnp.sum(xb * xb, axis=1, keepdims=True)
    sq_scr[...] = sq_col
    sq_row = sq_col.reshape(1, _N)

    d_scr[...] = jax.lax.dot_general(
        xb, xb, (((1,), (1,)), ((), ())),
        preferred_element_type=jnp.float32)

    n_chunks = _N // _RC
    col_iota = jax.lax.broadcasted_iota(jnp.int32, (_RC, _N), 1)

    def p1_body(c, d2_max):
        r0 = c * _RC
        sq_chunk = sq_scr[pl.ds(r0, _RC), :]
        dc = _d2(d_scr[pl.ds(r0, _RC), :], sq_chunk, sq_row)
        d2_max = jnp.maximum(d2_max, jnp.max(dc))

        def knn_body(_, carry):
            y, acc = carry
            m2 = jnp.min(y, axis=1, keepdims=True)
            am = jnp.min(jnp.where(y == m2, col_iota, _N),
                         axis=1, keepdims=True)
            m = jnp.sqrt(m2) / _SQRT_C
            acc = acc + m * m
            y = jnp.where(col_iota == am, jnp.float32(jnp.inf), y)
            return y, acc

        _, acc = jax.lax.fori_loop(
            0, _K, knn_body, (dc, jnp.zeros((_RC, 1), jnp.float32)))
        dens_scr[pl.ds(r0, _RC), :] = jnp.exp(-(acc / _K))
        return d2_max

    d2_max = jax.lax.fori_loop(0, n_chunks, p1_body, jnp.float32(-jnp.inf))
    dist_max = jnp.sqrt(d2_max) / _SQRT_C

    dens_scr[...] = dens_scr[...] + noise_ref[0].reshape(_N, 1)
    dens_row = jnp.transpose(dens_scr[...])

    def p2_body(c, _):
        r0 = c * _RC
        sq_chunk = sq_scr[pl.ds(r0, _RC), :]
        dc = _dist(d_scr[pl.ds(r0, _RC), :], sq_chunk, sq_row)
        dens_chunk = dens_scr[pl.ds(r0, _RC), :]
        cond = dens_row > dens_chunk
        delta = jnp.min(jnp.where(cond, dc, dist_max), axis=1, keepdims=True)
        score_scr[pl.ds(c, 1), :] = jnp.transpose(delta * dens_chunk)
        return 0

    jax.lax.fori_loop(0, n_chunks, p2_body, 0)

    flat_iota = jax.lax.broadcasted_iota(jnp.int32, (n_chunks, _RC), 0) * _RC \
        + jax.lax.broadcasted_iota(jnp.int32, (n_chunks, _RC), 1)
    lane_iota = jax.lax.broadcasted_iota(jnp.int32, (1, _N), 1)

    def topk_body(j, carry):
        sc, rank_row = carry
        m = jnp.max(sc)
        idx = jnp.min(jnp.where(sc == m, flat_iota, _N))
        rank_row = jnp.where(lane_iota == idx, j, rank_row)
        sc = jnp.where(flat_iota == idx, -jnp.float32(jnp.inf), sc)
        return sc, rank_row

    rank0 = jnp.full((1, _N), _CN, jnp.int32)
    _, rank_row = jax.lax.fori_loop(0, _CN, topk_body, (score_scr[...], rank0))

    riota = jax.lax.broadcasted_iota(jnp.int32, (_CN, _N), 0)
    p_oh = (riota == rank_row).astype(jnp.float32)
    dg = jax.lax.dot_general(
        p_oh, d_scr[...], (((1,), (0,)), ((), ())),
        preferred_element_type=jnp.float32)
    sqc = jax.lax.dot_general(
        p_oh, sq_scr[...], (((1,), (0,)), ((), ())),
        preferred_element_type=jnp.float32)
    dist_g = _dist(dg, sqc, sq_row)
    dmin = jnp.min(dist_g, axis=0, keepdims=True)
    bj = jnp.min(jnp.where(dist_g == dmin, riota, _CN), axis=0, keepdims=True)
    clu = jnp.where(rank_row < _CN, rank_row, bj)

    wt_row = jnp.exp(jax.lax.dot_general(
        w_ref[...], xb, (((1,), (1,)), ((), ())),
        preferred_element_type=jnp.float32) + b_ref[0, 0])
    at = (riota == clu).astype(jnp.float32)
    aw = jax.lax.dot_general(
        wt_row, at, (((1,), (1,)), ((), ())),
        preferred_element_type=jnp.float32) + 1e-6
    aw_tok = jax.lax.dot_general(
        aw, at, (((1,), (0,)), ((), ())),
        preferred_element_type=jnp.float32)
    norm = wt_row / aw_tok
    out_ref[0] = jax.lax.dot_general(
        at * norm, xb, (((1,), (0,)), ((), ())),
        preferred_element_type=jnp.float32)


def kernel(x, W_score, b_score):
    noise = jax.random.uniform(jax.random.key(1), (_B, _N),
                               dtype=jnp.float32) * 1e-06
    return pl.pallas_call(
        _ctm_kernel,
        grid=(_B,),
        in_specs=[
            pl.BlockSpec((1, _N, _C), lambda i: (i, 0, 0)),
            pl.BlockSpec((1, _C), lambda i: (0, 0)),
            pl.BlockSpec((1, 1), lambda i: (0, 0)),
            pl.BlockSpec((1, 1, _N), lambda i: (i, 0, 0)),
        ],
        compiler_params=pltpu.CompilerParams(
            dimension_semantics=("parallel",)),
        out_specs=pl.BlockSpec((1, _CN, _C), lambda i: (i, 0, 0)),
        out_shape=jax.ShapeDtypeStruct((_B, _CN, _C), jnp.float32),
        scratch_shapes=[
            pltpu.VMEM((_N, _N), jnp.float32),
            pltpu.VMEM((_N, 1), jnp.float32),
            pltpu.VMEM((_N, 1), jnp.float32),
            pltpu.VMEM((_N // _RC, _RC), jnp.float32),
        ],
    )(x, W_score, b_score.reshape(1, 1), noise.reshape(_B, 1, _N))
